# baseline (speedup 1.0000x reference)
import functools
import math

import jax
import jax.numpy as jnp
from jax import lax
from jax.experimental import pallas as pl
from jax.experimental.pallas import tpu as pltpu
from jax.experimental.pallas import tpu_sc as plsc

_LANES = 16
_IDX_CHUNK = 128


@functools.partial(jax.jit, static_argnums=(3, 4, 5))
def _embed_lookup(idx_flat, tok2, pos2, n_rows, seq_len, scale):
    d2 = tok2.shape[1]
    d = d2 // 2
    info = plsc.get_sparse_core_info()
    n_workers = info.num_cores * info.num_subcores
    per_w = n_rows // n_workers
    out_rows_w = per_w // 2
    chunks = per_w // _IDX_CHUNK
    mesh = plsc.VectorSubcoreMesh(core_axis_name="c", subcore_axis_name="s")

    @functools.partial(
        pl.kernel,
        mesh=mesh,
        out_type=jax.ShapeDtypeStruct((n_rows // 2, d2), jnp.float32),
        scratch_types=[
            pltpu.VMEM((per_w + _LANES,), jnp.int32),
            pltpu.VMEM((per_w,), jnp.int32),
            pltpu.VMEM((per_w, d2), jnp.float32),
            pltpu.VMEM((out_rows_w, d2), jnp.float32),
            pltpu.VMEM((out_rows_w, d2), jnp.float32),
            pltpu.SemaphoreType.DMA,
            pltpu.SemaphoreType.DMA,
        ],
    )
    def run(idx_hbm, tok_hbm, pos_hbm, out_hbm,
            idx_v, hi_v, rows_v, pos_v, out_v, gsem, psem):
        wid = lax.axis_index("s") * info.num_cores + lax.axis_index("c")
        base = wid * per_w
        pos_row0 = lax.rem(wid * out_rows_w, seq_len // 2)
        pos_copy = pltpu.async_copy(
            pos_hbm.at[pl.ds(pos_row0, out_rows_w)], pos_v, psem)
        pltpu.sync_copy(idx_hbm.at[pl.ds(base, per_w)],
                        idx_v.at[pl.ds(0, per_w)])

        def halve(i, _):
            sl = pl.ds(i * _LANES, _LANES)
            hi_v[sl] = lax.shift_right_logical(idx_v[sl], 1)
            return ()

        lax.fori_loop(0, per_w // _LANES, halve, (), unroll=4)

        gathers = []
        for j in range(chunks):
            gathers.append(pltpu.async_copy(
                tok_hbm.at[hi_v.at[pl.ds(j * _IDX_CHUNK, _IDX_CHUNK)]],
                rows_v.at[pl.ds(j * _IDX_CHUNK, _IDX_CHUNK)],
                gsem))
        for g in gathers:
            g.wait()
        pos_copy.wait()

        def body(j, _):
            for u in range(2):
                r = 2 * j + u
                iv = idx_v[pl.ds(r, _LANES)]
                half = lax.rem(iv[0], 2) * d
                for c in range(d // _LANES):
                    src = pl.ds(half + c * _LANES, _LANES)
                    dst = pl.ds(u * d + c * _LANES, _LANES)
                    out_v[j, dst] = rows_v[r, src] * scale + pos_v[j, dst]
            return ()

        lax.fori_loop(0, out_rows_w, body, (), unroll=2)
        pltpu.sync_copy(out_v, out_hbm.at[pl.ds(wid * out_rows_w, out_rows_w)])

    return run(idx_flat, tok2, pos2)


def kernel(inputs, token_table, pos_table):
    b, s = inputs.shape
    v, d = token_table.shape
    n = b * s
    scale = float(math.sqrt(d))
    idx_flat = inputs.reshape(n).astype(jnp.int32)
    tok2 = token_table.reshape(v // 2, 2 * d)
    pos2 = pos_table[:s].reshape(s // 2, 2 * d)
    out2 = _embed_lookup(idx_flat, tok2, pos2, n, s, scale)
    return out2.reshape(b, s, d)


# speedup vs baseline: 2.5441x; 2.5441x over previous
import functools
import math

import jax
import jax.numpy as jnp
from jax import lax
from jax.experimental import pallas as pl
from jax.experimental.pallas import tpu as pltpu
from jax.experimental.pallas import tpu_sc as plsc

_LANES = 16


@functools.partial(jax.jit, static_argnums=(3, 4))
def _embed_lookup_t(inputs, tok_t, pos_t, n_batch, scale):
    d, v = tok_t.shape
    b, s = inputs.shape
    n = b * s
    info = plsc.get_sparse_core_info()
    n_workers = info.num_cores * info.num_subcores
    dims_per_w = d // n_workers
    chunks_per_batch = s // _LANES
    mesh = plsc.VectorSubcoreMesh(core_axis_name="c", subcore_axis_name="s")

    @functools.partial(
        pl.kernel,
        mesh=mesh,
        compiler_params=pltpu.CompilerParams(needs_layout_passes=False),
        out_type=jax.ShapeDtypeStruct((b, d, s), jnp.float32),
        scratch_types=[
            pltpu.VMEM((n,), jnp.int32),
            pltpu.VMEM((v,), jnp.float32),
            pltpu.VMEM((s,), jnp.float32),
            pltpu.VMEM((n,), jnp.float32),
            pltpu.SemaphoreType.DMA,
        ],
    )
    def run(idx_hbm, tok_hbm, pos_hbm, out_hbm, idx_v, row_v, pos_v, out_v, isem):
        wid = lax.axis_index("s") * info.num_cores + lax.axis_index("c")
        idx_copies = [
            pltpu.async_copy(idx_hbm.at[bb],
                             idx_v.at[pl.ds(bb * s, s)], isem)
            for bb in range(b)
        ]
        for j in range(dims_per_w):
            dim = wid * dims_per_w + j
            pltpu.sync_copy(pos_hbm.at[dim, pl.ds(0, s)], pos_v)
            pltpu.sync_copy(tok_hbm.at[dim], row_v)
            if j == 0:
                for c in idx_copies:
                    c.wait()

            def body(k, _):
                sl = pl.ds(k * _LANES, _LANES)
                iv = idx_v[sl]
                g = plsc.load_gather(row_v, [iv])
                p = pos_v[pl.ds(lax.rem(k, chunks_per_batch) * _LANES, _LANES)]
                out_v[sl] = g * scale + p
                return ()

            lax.fori_loop(0, n // _LANES, body, (), unroll=4)
            for bb in range(b):
                pltpu.sync_copy(out_v.at[pl.ds(bb * s, s)],
                                out_hbm.at[bb, dim, pl.ds(0, s)])

    return run(inputs, tok_t, pos_t)


def kernel(inputs, token_table, pos_table):
    b, s = inputs.shape
    d = token_table.shape[1]
    scale = float(math.sqrt(d))
    out_t = _embed_lookup_t(inputs.astype(jnp.int32), token_table.T,
                            pos_table.T, b, scale)
    return out_t.transpose(0, 2, 1)
